# Initial kernel scaffold; baseline (speedup 1.0000x reference)
import jax
import jax.numpy as jnp
from jax.experimental import pallas as pl


def kernel(indices, G0, G1, G2):
    raise NotImplementedError("write your pallas kernel here")


# speedup vs baseline: 2.5739x; 2.5739x over previous
import functools

import jax
import jax.numpy as jnp
from jax import lax
from jax.experimental import pallas as pl
from jax.experimental.pallas import tpu as pltpu
from jax.experimental.pallas import tpu_sc as plsc

P0, P1, P2 = 100, 100, 100
Q0, Q1, Q2 = 4, 4, 4
R1, R2 = 32, 32
B = 16384
DT = Q1 * R2
TROW = Q0 * DT
NPAIR = P0 * P1

L = 16
NC = 2
NS = 16
NW = NC * NS
BPW = B // NW
CH = 64
NCH = BPW // CH
NG = CH // L
RC = 8

CI = 20


def _tc_pair_table(g0_ref, g1_ref, out_ref):
    g0 = g0_ref[...]
    for j in range(CI):
        mm = jnp.dot(g0, g1_ref[j], preferred_element_type=jnp.float32)
        out_ref[j] = mm.reshape(P0, Q0, DT)


def _sc_lookup_body(idx_hbm, t_hbm, g2t_hbm, out_hbm,
                    idxv, pairv, i2v, trows, crows, outv, sem1, sem2):
    wid = lax.axis_index("s") * NC + lax.axis_index("c")
    base = wid * BPW
    pltpu.sync_copy(idx_hbm.at[pl.ds(base, BPW)], idxv)
    lane = lax.iota(jnp.int32, L)

    def chunk_body(ci, carry):
        off = ci * CH
        for g in range(NG):
            v = idxv[pl.ds(off + g * L, L)]
            i0 = lax.div(v, P1 * P2)
            rem = v - i0 * (P1 * P2)
            i1 = lax.div(rem, P2)
            i2 = rem - i1 * P2
            pairv[pl.ds(g * L, L)] = i1 * P0 + i0
            i2v[pl.ds(g * L, L)] = i2
        cp1 = pltpu.async_copy(t_hbm.at[pairv], trows, sem1)
        cp2 = pltpu.async_copy(g2t_hbm.at[i2v], crows, sem2)
        cp1.wait()
        cp2.wait()

        def group_body(g, gcarry):
            rows = lane + g * L
            for rc in range(R2 // RC):
                cregs = [[plsc.load_gather(crows, [rows, jnp.full((L,), q2 * R2 + rc * RC + r, jnp.int32)])
                          for r in range(RC)] for q2 in range(Q2)]
                for q0 in range(Q0):
                    for q1 in range(Q1):
                        tbase = q0 * DT + q1 * R2 + rc * RC
                        tregs = [plsc.load_gather(trows, [rows, jnp.full((L,), tbase + r, jnp.int32)])
                                 for r in range(RC)]
                        for q2 in range(Q2):
                            s = tregs[0] * cregs[q2][0]
                            for r in range(1, RC):
                                s = s + tregs[r] * cregs[q2][r]
                            ocol = jnp.full((L,), q0 * 16 + q1 * 4 + q2, jnp.int32)
                            if rc == 0:
                                plsc.store_scatter(outv, [rows, ocol], s)
                            else:
                                plsc.addupdate_scatter(outv, [rows, ocol], s)
            return gcarry

        lax.fori_loop(0, NG, group_body, 0)
        pltpu.sync_copy(outv, out_hbm.at[pl.ds(base + off, CH)])
        return carry

    lax.fori_loop(0, NCH, chunk_body, 0)


def kernel(indices, G0, G1, G2):
    idx = indices.astype(jnp.int32)
    g0r = G0.reshape(P0 * Q0, R1)
    g1r = G1.reshape(P1, R1, DT)
    g2t = G2.reshape(P2, R2, Q2).transpose(0, 2, 1).reshape(P2, Q2 * R2)

    t4 = pl.pallas_call(
        _tc_pair_table,
        grid=(P1 // CI,),
        in_specs=[
            pl.BlockSpec((P0 * Q0, R1), lambda i: (0, 0)),
            pl.BlockSpec((CI, R1, DT), lambda i: (i, 0, 0)),
        ],
        out_specs=pl.BlockSpec((CI, P0, Q0, DT), lambda i: (i, 0, 0, 0)),
        out_shape=jax.ShapeDtypeStruct((P1, P0, Q0, DT), jnp.float32),
    )(g0r, g1r)
    t_table = t4.reshape(NPAIR, TROW)

    sc = functools.partial(
        pl.kernel,
        mesh=plsc.VectorSubcoreMesh(core_axis_name="c", subcore_axis_name="s"),
        out_type=jax.ShapeDtypeStruct((B, Q0 * Q1 * Q2), jnp.float32),
        compiler_params=pltpu.CompilerParams(needs_layout_passes=False),
        scratch_types=[
            pltpu.VMEM((BPW,), jnp.int32),
            pltpu.VMEM((CH,), jnp.int32),
            pltpu.VMEM((CH,), jnp.int32),
            pltpu.VMEM((CH, TROW), jnp.float32),
            pltpu.VMEM((CH, Q2 * R2), jnp.float32),
            pltpu.VMEM((CH, Q0 * Q1 * Q2), jnp.float32),
            pltpu.SemaphoreType.DMA,
            pltpu.SemaphoreType.DMA,
        ],
    )(_sc_lookup_body)
    return sc(idx, t_table, g2t)


# speedup vs baseline: 2.7104x; 1.0530x over previous
import functools

import jax
import jax.numpy as jnp
from jax import lax
from jax.experimental import pallas as pl
from jax.experimental.pallas import tpu as pltpu
from jax.experimental.pallas import tpu_sc as plsc

P0, P1, P2 = 100, 100, 100
Q0, Q1, Q2 = 4, 4, 4
R1, R2 = 32, 32
B = 16384
DT = Q1 * R2
TROW = Q0 * DT
NPAIR = P0 * P1

L = 16
NC = 2
NS = 16
NW = NC * NS
BPW = B // NW
CH = 64
NCH = BPW // CH
NG = CH // L
RC = 8

CI = 20


def _tc_pair_table(g0_ref, g1_ref, out_ref):
    for j in range(CI):
        g1 = g1_ref[j]
        for q0 in range(Q0):
            mm = jnp.dot(g0_ref[:, q0, :], g1, preferred_element_type=jnp.float32)
            out_ref[pl.ds(j * P0, P0), pl.ds(q0 * DT, DT)] = mm


def _sc_lookup_body(idx_hbm, t_hbm, g2t_hbm, out_hbm,
                    idxv, pairv, i2v, trows, crows, outv, sem1, sem2):
    wid = lax.axis_index("s") * NC + lax.axis_index("c")
    base = wid * BPW
    pltpu.sync_copy(idx_hbm.at[pl.ds(base, BPW)], idxv)
    lane = lax.iota(jnp.int32, L)

    def chunk_body(ci, carry):
        off = ci * CH
        for g in range(NG):
            v = idxv[pl.ds(off + g * L, L)]
            i0 = lax.div(v, P1 * P2)
            rem = v - i0 * (P1 * P2)
            i1 = lax.div(rem, P2)
            i2 = rem - i1 * P2
            pairv[pl.ds(g * L, L)] = i1 * P0 + i0
            i2v[pl.ds(g * L, L)] = i2
        cp1 = pltpu.async_copy(t_hbm.at[pairv], trows, sem1)
        cp2 = pltpu.async_copy(g2t_hbm.at[i2v], crows, sem2)
        cp1.wait()
        cp2.wait()

        def group_body(g, gcarry):
            rows = lane + g * L
            for rc in range(R2 // RC):
                cregs = [[plsc.load_gather(crows, [rows, jnp.full((L,), q2 * R2 + rc * RC + r, jnp.int32)])
                          for r in range(RC)] for q2 in range(Q2)]
                for q0 in range(Q0):
                    for q1 in range(Q1):
                        tbase = q0 * DT + q1 * R2 + rc * RC
                        tregs = [plsc.load_gather(trows, [rows, jnp.full((L,), tbase + r, jnp.int32)])
                                 for r in range(RC)]
                        for q2 in range(Q2):
                            s = tregs[0] * cregs[q2][0]
                            for r in range(1, RC):
                                s = s + tregs[r] * cregs[q2][r]
                            ocol = jnp.full((L,), q0 * 16 + q1 * 4 + q2, jnp.int32)
                            if rc == 0:
                                plsc.store_scatter(outv, [rows, ocol], s)
                            else:
                                plsc.addupdate_scatter(outv, [rows, ocol], s)
            return gcarry

        lax.fori_loop(0, NG, group_body, 0)
        pltpu.sync_copy(outv, out_hbm.at[pl.ds(base + off, CH)])
        return carry

    lax.fori_loop(0, NCH, chunk_body, 0)


def kernel(indices, G0, G1, G2):
    idx = indices.astype(jnp.int32)
    g0q = G0.reshape(P0, Q0, R1)
    g1r = G1.reshape(P1, R1, DT)
    g2t = G2.reshape(P2, R2, Q2).transpose(0, 2, 1).reshape(P2, Q2 * R2)

    t_table = pl.pallas_call(
        _tc_pair_table,
        grid=(P1 // CI,),
        in_specs=[
            pl.BlockSpec((P0, Q0, R1), lambda i: (0, 0, 0)),
            pl.BlockSpec((CI, R1, DT), lambda i: (i, 0, 0)),
        ],
        out_specs=pl.BlockSpec((CI * P0, TROW), lambda i: (i, 0)),
        out_shape=jax.ShapeDtypeStruct((NPAIR, TROW), jnp.float32),
    )(g0q, g1r)

    sc = functools.partial(
        pl.kernel,
        mesh=plsc.VectorSubcoreMesh(core_axis_name="c", subcore_axis_name="s"),
        out_type=jax.ShapeDtypeStruct((B, Q0 * Q1 * Q2), jnp.float32),
        compiler_params=pltpu.CompilerParams(
            needs_layout_passes=False, disable_bounds_checks=True),
        scratch_types=[
            pltpu.VMEM((BPW,), jnp.int32),
            pltpu.VMEM((CH,), jnp.int32),
            pltpu.VMEM((CH,), jnp.int32),
            pltpu.VMEM((CH, TROW), jnp.float32),
            pltpu.VMEM((CH, Q2 * R2), jnp.float32),
            pltpu.VMEM((CH, Q0 * Q1 * Q2), jnp.float32),
            pltpu.SemaphoreType.DMA,
            pltpu.SemaphoreType.DMA,
        ],
    )(_sc_lookup_body)
    return sc(idx, t_table, g2t)
